# baseline (speedup 1.0000x reference)
import functools

import jax
import jax.numpy as jnp
from jax import lax
from jax.experimental import pallas as pl
from jax.experimental.pallas import tpu as pltpu
from jax.experimental.pallas import tpu_sc as plsc

NEG = -1000000.0
NUM_WORKERS = 32


def _gather_rows(ht, r, ent_emb, ling, visual, rel_emb, *, interpret=False):
    nht_total = ht.shape[0]
    nr_total = r.shape[0]
    d = ent_emb.shape[1]
    nht = nht_total // NUM_WORKERS
    nr = nr_total // NUM_WORKERS
    mesh = plsc.VectorSubcoreMesh(core_axis_name="c", subcore_axis_name="s")

    @functools.partial(
        pl.kernel,
        out_type=[jax.ShapeDtypeStruct((nht_total, d), jnp.float32)] * 3
        + [jax.ShapeDtypeStruct((nr_total, d), jnp.float32)],
        mesh=mesh,
        scratch_types=[
            pltpu.VMEM((nht,), jnp.int32),
            pltpu.VMEM((nr,), jnp.int32),
            pltpu.VMEM((nht, d), jnp.float32),
            pltpu.VMEM((nr, d), jnp.float32),
            pltpu.SemaphoreType.DMA,
        ],
        interpret=interpret,
    )
    def k(ht_hbm, r_hbm, ent_hbm, ling_hbm, vis_hbm, rel_hbm,
          oe, ol, ov, orel, idxh, idxr, bufh, bufr, sem):
        wid = lax.axis_index("s") * 2 + lax.axis_index("c")
        bh = wid * nht
        br = wid * nr
        pltpu.sync_copy(ht_hbm.at[pl.ds(bh, nht)], idxh)
        pltpu.sync_copy(r_hbm.at[pl.ds(br, nr)], idxr)
        for table, out in ((ent_hbm, oe), (ling_hbm, ol), (vis_hbm, ov)):
            pltpu.async_copy(table.at[idxh], bufh, sem).wait()
            pltpu.sync_copy(bufh, out.at[pl.ds(bh, nht)])
        pltpu.async_copy(rel_hbm.at[idxr], bufr, sem).wait()
        pltpu.sync_copy(bufr, orel.at[pl.ds(br, nr)])

    return k(ht, r, ent_emb, ling, visual, rel_emb)


def _prep_body(er, lr, vr, rr, wl, wv, q_ref, tgt_ref):
    b = rr.shape[0]
    fused = (
        er[...]
        + jnp.dot(lr[...], wl[...], preferred_element_type=jnp.float32)
        + jnp.dot(vr[...], wv[...], preferred_element_type=jnp.float32)
    )
    q = fused[:b, :] * rr[...]
    q_ref[...] = q
    tgt_ref[...] = jnp.sum(q * fused[b:, :], axis=1, keepdims=True)


def _prep_call(er, lr, vr, rr, wl, wv, *, interpret=False):
    b, d = rr.shape
    return pl.pallas_call(
        _prep_body,
        out_shape=[
            jax.ShapeDtypeStruct((b, d), jnp.float32),
            jax.ShapeDtypeStruct((b, 1), jnp.float32),
        ],
        interpret=interpret,
    )(er, lr, vr, rr, wl, wv)


def _main_body(c_tile, n, q, tgt, ent, lingb, visb, wl, wv, scores_ref, cnt_ref):
    j = pl.program_id(0)
    ae = (
        ent[...]
        + jnp.dot(lingb[...], wl[...], preferred_element_type=jnp.float32)
        + jnp.dot(visb[...], wv[...], preferred_element_type=jnp.float32)
    )
    s = lax.dot_general(q[...], ae, (((1,), (1,)), ((), ())),
                        preferred_element_type=jnp.float32)
    scores_ref[...] = s
    col = j * c_tile + lax.broadcasted_iota(jnp.int32, s.shape, 1)
    ge = jnp.where((s >= tgt[...]) & (col < n), 1.0, 0.0)

    @pl.when(j == 0)
    def _():
        cnt_ref[...] = jnp.zeros_like(cnt_ref)

    cnt_ref[...] += jnp.sum(ge, axis=1, keepdims=True)


def _main_call(q, tgt, ent, ling, vis, wl, wv, *, c_tile=1024, interpret=False):
    b, d = q.shape
    n = ent.shape[0]
    grid = (pl.cdiv(n, c_tile),)
    return pl.pallas_call(
        functools.partial(_main_body, c_tile, n),
        grid=grid,
        in_specs=[
            pl.BlockSpec((b, d), lambda j: (0, 0)),
            pl.BlockSpec((b, 1), lambda j: (0, 0)),
            pl.BlockSpec((c_tile, d), lambda j: (j, 0)),
            pl.BlockSpec((c_tile, d), lambda j: (j, 0)),
            pl.BlockSpec((c_tile, d), lambda j: (j, 0)),
            pl.BlockSpec((d, d), lambda j: (0, 0)),
            pl.BlockSpec((d, d), lambda j: (0, 0)),
        ],
        out_specs=[
            pl.BlockSpec((b, c_tile), lambda j: (0, j)),
            pl.BlockSpec((b, 1), lambda j: (0, 0)),
        ],
        out_shape=[
            jax.ShapeDtypeStruct((b, n), jnp.float32),
            jax.ShapeDtypeStruct((b, 1), jnp.float32),
        ],
        interpret=interpret,
    )(q, tgt, ent, ling, vis, wl, wv)


def _filter_copy(scores, idx64, *, interpret=False):
    b, n = scores.shape
    kk = idx64.shape[1]
    nrows = b // NUM_WORKERS
    ngrp = kk // 16
    mesh = plsc.VectorSubcoreMesh(core_axis_name="c", subcore_axis_name="s")

    @functools.partial(
        pl.kernel,
        out_type=[
            jax.ShapeDtypeStruct((b, n), jnp.float32),
            jax.ShapeDtypeStruct((b, kk), jnp.float32),
            jax.ShapeDtypeStruct((b, kk), jnp.float32),
        ],
        mesh=mesh,
        scratch_types=[
            pltpu.VMEM((n,), jnp.float32),
            pltpu.VMEM((n,), jnp.float32),
            pltpu.VMEM((kk,), jnp.int32),
            pltpu.VMEM((kk,), jnp.float32),
            pltpu.VMEM((kk,), jnp.float32),
            pltpu.SemaphoreType.DMA,
            pltpu.SemaphoreType.DMA,
        ],
        compiler_params=pltpu.CompilerParams(needs_layout_passes=False,
                                             use_tc_tiling_on_sc=True),
        interpret=interpret,
    )
    def k(scores_hbm, idx_hbm, filt_hbm, vals_hbm, g2_hbm,
          row0, row1, idxv, valv, g2v, insem, outsem):
        wid = lax.axis_index("s") * 2 + lax.axis_index("c")
        base = wid * nrows
        bufs = (row0, row1)
        out_dmas = [None, None]
        in_dma = pltpu.async_copy(scores_hbm.at[base], row0, insem)
        for step in range(nrows):
            slot = step % 2
            row = base + step
            rb = bufs[slot]
            in_dma.wait()
            if step + 1 < nrows:
                nslot = 1 - slot
                if out_dmas[nslot] is not None:
                    out_dmas[nslot].wait()
                    out_dmas[nslot] = None
                in_dma = pltpu.async_copy(
                    scores_hbm.at[row + 1], bufs[nslot], insem)
            pltpu.sync_copy(idx_hbm.at[row], idxv)
            for g in range(ngrp):
                sl = pl.ds(g * 16, 16)
                valv[sl] = plsc.load_gather(rb, [idxv[sl]])
            for g in range(ngrp):
                sl = pl.ds(g * 16, 16)
                mk = (lax.iota(jnp.int32, 16) + g * 16).astype(jnp.float32)
                plsc.store_scatter(rb, [idxv[sl]], mk)
            for g in range(ngrp):
                sl = pl.ds(g * 16, 16)
                g2v[sl] = plsc.load_gather(rb, [idxv[sl]])
            for g in range(ngrp):
                sl = pl.ds(g * 16, 16)
                plsc.store_scatter(rb, [idxv[sl]],
                                   jnp.full((16,), NEG, jnp.float32))
            pltpu.sync_copy(valv, vals_hbm.at[row])
            pltpu.sync_copy(g2v, g2_hbm.at[row])
            out_dmas[slot] = pltpu.async_copy(rb, filt_hbm.at[row], outsem)
        for od in out_dmas:
            if od is not None:
                od.wait()

    return k(scores, idx64)


def _finalize_body(vals, g2, tgt, cnt, ranks_ref):
    b, kk = vals.shape
    m = lax.broadcasted_iota(jnp.int32, (b, kk), 1).astype(jnp.float32)
    winner = g2[...] == m
    cmp = jnp.where(vals[...] >= tgt[...], 1.0, 0.0)
    negcmp = jnp.where(NEG >= tgt[...], 1.0, 0.0)
    corr = jnp.sum(jnp.where(winner, cmp - negcmp, 0.0), axis=1, keepdims=True)
    ranks_ref[...] = 1.0 + cnt[...] - corr


def _finalize_call(vals, g2, tgt, cnt, *, interpret=False):
    b = vals.shape[0]
    return pl.pallas_call(
        _finalize_body,
        out_shape=jax.ShapeDtypeStruct((b, 1), jnp.float32),
        interpret=interpret,
    )(vals, g2, tgt, cnt)


def kernel(queries, ling, visual, filter_idx, ent_emb, rel_emb, W_ling, W_visual):
    b = queries.shape[0]
    n, d = ent_emb.shape
    f = filter_idx.shape[1]
    h = queries[:, 0].astype(jnp.int32)
    r = queries[:, 1].astype(jnp.int32)
    t = queries[:, 2].astype(jnp.int32)
    ht = jnp.concatenate([h, t])

    er, lr, vr, rr = _gather_rows(ht, r, ent_emb, ling, visual, rel_emb)
    q, tgt = _prep_call(er, lr, vr, rr, W_ling, W_visual)
    scores, cnt = _main_call(q, tgt, ent_emb, ling, visual, W_ling, W_visual)

    kk = -(-(f + 1) // 16) * 16
    idx64 = jnp.concatenate(
        [filter_idx.astype(jnp.int32),
         jnp.broadcast_to(t[:, None], (b, kk - f))], axis=1)

    filtered, vals, g2 = _filter_copy(scores, idx64)
    ranks2 = _finalize_call(vals, g2, tgt, cnt)
    return filtered, tgt, ranks2.reshape(b)


# speedup vs baseline: 1.2313x; 1.2313x over previous
import functools

import jax
import jax.numpy as jnp
from jax import lax
from jax.experimental import pallas as pl
from jax.experimental.pallas import tpu as pltpu
from jax.experimental.pallas import tpu_sc as plsc

NEG = -1000000.0
NUM_WORKERS = 32


def _gather_rows(ht, r, ent_emb, ling, visual, rel_emb, *, interpret=False):
    nht_total = ht.shape[0]
    nr_total = r.shape[0]
    d = ent_emb.shape[1]
    nht = nht_total // NUM_WORKERS
    nr = nr_total // NUM_WORKERS
    mesh = plsc.VectorSubcoreMesh(core_axis_name="c", subcore_axis_name="s")

    @functools.partial(
        pl.kernel,
        out_type=[jax.ShapeDtypeStruct((nht_total, d), jnp.float32)] * 3
        + [jax.ShapeDtypeStruct((nr_total, d), jnp.float32)],
        mesh=mesh,
        scratch_types=[
            pltpu.VMEM((nht,), jnp.int32),
            pltpu.VMEM((nr,), jnp.int32),
            pltpu.VMEM((nht, d), jnp.float32),
            pltpu.VMEM((nr, d), jnp.float32),
            pltpu.SemaphoreType.DMA,
        ],
        interpret=interpret,
    )
    def k(ht_hbm, r_hbm, ent_hbm, ling_hbm, vis_hbm, rel_hbm,
          oe, ol, ov, orel, idxh, idxr, bufh, bufr, sem):
        wid = lax.axis_index("s") * 2 + lax.axis_index("c")
        bh = wid * nht
        br = wid * nr
        pltpu.sync_copy(ht_hbm.at[pl.ds(bh, nht)], idxh)
        pltpu.sync_copy(r_hbm.at[pl.ds(br, nr)], idxr)
        for table, out in ((ent_hbm, oe), (ling_hbm, ol), (vis_hbm, ov)):
            pltpu.async_copy(table.at[idxh], bufh, sem).wait()
            pltpu.sync_copy(bufh, out.at[pl.ds(bh, nht)])
        pltpu.async_copy(rel_hbm.at[idxr], bufr, sem).wait()
        pltpu.sync_copy(bufr, orel.at[pl.ds(br, nr)])

    return k(ht, r, ent_emb, ling, visual, rel_emb)


def _prep_body(er, lr, vr, rr, wl, wv, q_ref, tgt_ref):
    b = rr.shape[0]
    fused = (
        er[...]
        + jnp.dot(lr[...], wl[...], preferred_element_type=jnp.float32)
        + jnp.dot(vr[...], wv[...], preferred_element_type=jnp.float32)
    )
    q = fused[:b, :] * rr[...]
    q_ref[...] = q
    tgt_ref[...] = jnp.sum(q * fused[b:, :], axis=1, keepdims=True)


def _prep_call(er, lr, vr, rr, wl, wv, *, interpret=False):
    b, d = rr.shape
    return pl.pallas_call(
        _prep_body,
        out_shape=[
            jax.ShapeDtypeStruct((b, d), jnp.float32),
            jax.ShapeDtypeStruct((b, 1), jnp.float32),
        ],
        interpret=interpret,
    )(er, lr, vr, rr, wl, wv)


def _main_body(c_tile, n, q, tgt, ent, lingb, visb, wl, wv, mw, filt_ref, rank_ref):
    j = pl.program_id(0)
    b = q.shape[0]
    ae = (
        ent[...]
        + jnp.dot(lingb[...], wl[...], preferred_element_type=jnp.float32)
        + jnp.dot(visb[...], wv[...], preferred_element_type=jnp.float32)
    )
    s = lax.dot_general(q[...], ae, (((1,), (1,)), ((), ())),
                        preferred_element_type=jnp.float32)
    mw0 = mw[...][0]
    base_p = 8 * lax.rem(j, 4)
    acc = jnp.zeros((b, 1), jnp.float32)
    for qq in range(c_tile // 128):
        sl = slice(qq * 128, (qq + 1) * 128)
        cm = (jnp.right_shift(mw0, base_p + qq) & 1) == 1
        sc = jnp.where(cm, NEG, s[:, sl])
        filt_ref[:, sl] = sc
        col = j * c_tile + qq * 128 + lax.broadcasted_iota(
            jnp.int32, (b, 128), 1)
        ge = jnp.where((sc >= tgt[...]) & (col < n), 1.0, 0.0)
        acc += jnp.sum(ge, axis=1, keepdims=True)

    @pl.when(j == 0)
    def _():
        rank_ref[...] = jnp.ones_like(rank_ref)

    rank_ref[...] += acc


def _main_call(q, tgt, ent, ling, vis, wl, wv, maskw, *, c_tile=1024,
               interpret=False):
    b, d = q.shape
    n = ent.shape[0]
    grid = (pl.cdiv(n, c_tile),)
    return pl.pallas_call(
        functools.partial(_main_body, c_tile, n),
        grid=grid,
        in_specs=[
            pl.BlockSpec((b, d), lambda j: (0, 0)),
            pl.BlockSpec((b, 1), lambda j: (0, 0)),
            pl.BlockSpec((c_tile, d), lambda j: (j, 0)),
            pl.BlockSpec((c_tile, d), lambda j: (j, 0)),
            pl.BlockSpec((c_tile, d), lambda j: (j, 0)),
            pl.BlockSpec((d, d), lambda j: (0, 0)),
            pl.BlockSpec((d, d), lambda j: (0, 0)),
            pl.BlockSpec((1, b, 128), lambda j: (j // 4, 0, 0)),
        ],
        out_specs=[
            pl.BlockSpec((b, c_tile), lambda j: (0, j)),
            pl.BlockSpec((b, 1), lambda j: (0, 0)),
        ],
        out_shape=[
            jax.ShapeDtypeStruct((b, n), jnp.float32),
            jax.ShapeDtypeStruct((b, 1), jnp.float32),
        ],
        interpret=interpret,
    )(q, tgt, ent, ling, vis, wl, wv, maskw)


def _mask_build(idx64, n_cols, *, interpret=False):
    b, kk = idx64.shape
    ngrp = kk // 16
    nrows = b // NUM_WORKERS
    ntiles = n_cols // 4096
    mesh = plsc.VectorSubcoreMesh(core_axis_name="c", subcore_axis_name="s")

    @functools.partial(
        pl.kernel,
        out_type=jax.ShapeDtypeStruct((ntiles, b, 128), jnp.int32),
        mesh=mesh,
        scratch_types=[
            pltpu.VMEM((n_cols,), jnp.int32),
            pltpu.VMEM((ntiles, 128), jnp.int32),
            pltpu.VMEM((kk,), jnp.int32),
        ],
        compiler_params=pltpu.CompilerParams(needs_layout_passes=False),
        interpret=interpret,
    )
    def k(idx_hbm, mask_hbm, elect, mrow, idxv):
        wid = lax.axis_index("s") * 2 + lax.axis_index("c")
        base = wid * nrows
        for step in range(nrows):
            row = base + step
            pltpu.sync_copy(idx_hbm.at[row], idxv)
            for g in range(ngrp):
                sl = pl.ds(g * 16, 16)
                ent = lax.iota(jnp.int32, 16) + g * 16
                plsc.store_scatter(elect, [idxv[sl]], ent)
            zero = jnp.zeros((16,), jnp.int32)
            for g in range(ntiles * 128 // 16):
                mrow[g * 16 // 128, pl.ds(g * 16 % 128, 16)] = zero
            for g in range(ngrp):
                sl = pl.ds(g * 16, 16)
                ent = lax.iota(jnp.int32, 16) + g * 16
                ii = idxv[sl]
                win = plsc.load_gather(elect, [ii]) == ent
                jhi = lax.shift_right_logical(ii, 12)
                ll = ii & 127
                p = lax.shift_right_logical(ii & 4095, 7)
                bit = lax.shift_left(jnp.ones((16,), jnp.int32), p)
                plsc.addupdate_scatter(mrow, [jhi, ll], bit, mask=win)
            pltpu.sync_copy(mrow, mask_hbm.at[:, row])

    return k(idx64)


def kernel(queries, ling, visual, filter_idx, ent_emb, rel_emb, W_ling, W_visual):
    b = queries.shape[0]
    n, d = ent_emb.shape
    f = filter_idx.shape[1]
    h = queries[:, 0].astype(jnp.int32)
    r = queries[:, 1].astype(jnp.int32)
    t = queries[:, 2].astype(jnp.int32)
    ht = jnp.concatenate([h, t])

    kk = -(-(f + 1) // 16) * 16
    idx64 = jnp.concatenate(
        [filter_idx.astype(jnp.int32),
         jnp.broadcast_to(t[:, None], (b, kk - f))], axis=1)

    c_tile = 1024
    n_pad4 = pl.cdiv(n, 4096) * 4096
    maskw = _mask_build(idx64, n_pad4)

    er, lr, vr, rr = _gather_rows(ht, r, ent_emb, ling, visual, rel_emb)
    q, tgt = _prep_call(er, lr, vr, rr, W_ling, W_visual)
    filtered, ranks2 = _main_call(q, tgt, ent_emb, ling, visual,
                                  W_ling, W_visual, maskw, c_tile=c_tile)
    return filtered, tgt, ranks2.reshape(b)


# speedup vs baseline: 2.5461x; 2.0677x over previous
import functools

import jax
import jax.numpy as jnp
from jax import lax
from jax.experimental import pallas as pl
from jax.experimental.pallas import tpu as pltpu
from jax.experimental.pallas import tpu_sc as plsc

NEG = -1000000.0
NUM_WORKERS = 32


def _gather_rows(ht, r, ent_emb, ling, visual, rel_emb, *, interpret=False):
    nht_total = ht.shape[0]
    nr_total = r.shape[0]
    d = ent_emb.shape[1]
    nht = nht_total // NUM_WORKERS
    nr = nr_total // NUM_WORKERS
    mesh = plsc.VectorSubcoreMesh(core_axis_name="c", subcore_axis_name="s")

    @functools.partial(
        pl.kernel,
        out_type=[jax.ShapeDtypeStruct((nht_total, d), jnp.float32)] * 3
        + [jax.ShapeDtypeStruct((nr_total, d), jnp.float32)],
        mesh=mesh,
        scratch_types=[
            pltpu.VMEM((nht,), jnp.int32),
            pltpu.VMEM((nr,), jnp.int32),
            pltpu.VMEM((nht, d), jnp.float32),
            pltpu.VMEM((nr, d), jnp.float32),
            pltpu.SemaphoreType.DMA,
        ],
        interpret=interpret,
    )
    def k(ht_hbm, r_hbm, ent_hbm, ling_hbm, vis_hbm, rel_hbm,
          oe, ol, ov, orel, idxh, idxr, bufh, bufr, sem):
        wid = lax.axis_index("s") * 2 + lax.axis_index("c")
        bh = wid * nht
        br = wid * nr
        pltpu.sync_copy(ht_hbm.at[pl.ds(bh, nht)], idxh)
        pltpu.sync_copy(r_hbm.at[pl.ds(br, nr)], idxr)
        for table, out in ((ent_hbm, oe), (ling_hbm, ol), (vis_hbm, ov)):
            pltpu.async_copy(table.at[idxh], bufh, sem).wait()
            pltpu.sync_copy(bufh, out.at[pl.ds(bh, nht)])
        pltpu.async_copy(rel_hbm.at[idxr], bufr, sem).wait()
        pltpu.sync_copy(bufr, orel.at[pl.ds(br, nr)])

    return k(ht, r, ent_emb, ling, visual, rel_emb)


def _prep_body(er, lr, vr, rr, wl, wv, q_ref, tgt_ref):
    b = rr.shape[0]
    fused = (
        er[...]
        + jnp.dot(lr[...], wl[...], preferred_element_type=jnp.float32)
        + jnp.dot(vr[...], wv[...], preferred_element_type=jnp.float32)
    )
    q = fused[:b, :] * rr[...]
    q_ref[...] = q
    tgt_ref[...] = jnp.sum(q * fused[b:, :], axis=1)[None, :]


def _prep_call(er, lr, vr, rr, wl, wv, *, interpret=False):
    b, d = rr.shape
    return pl.pallas_call(
        _prep_body,
        out_shape=[
            jax.ShapeDtypeStruct((b, d), jnp.float32),
            jax.ShapeDtypeStruct((1, b), jnp.float32),
        ],
        interpret=interpret,
    )(er, lr, vr, rr, wl, wv)


def _main_body(c_tile, n, q, tgt, ent, lingb, visb, wl, wv, mw, filt_ref, rank_ref):
    j = pl.program_id(0)
    b = q.shape[0]
    ae = (
        ent[...]
        + jnp.dot(lingb[...], wl[...], preferred_element_type=jnp.float32)
        + jnp.dot(visb[...], wv[...], preferred_element_type=jnp.float32)
    )
    s = lax.dot_general(ae, q[...], (((1,), (1,)), ((), ())),
                        preferred_element_type=jnp.float32)
    mw0 = mw[...][0]
    base_p = 8 * lax.rem(j, 4)
    acc = jnp.zeros((1, b), jnp.float32)
    for qq in range(c_tile // 128):
        sl = slice(qq * 128, (qq + 1) * 128)
        cm = (jnp.right_shift(mw0, base_p + qq) & 1) == 1
        sc = jnp.where(cm, NEG, s[sl, :])
        filt_ref[sl, :] = sc
        col = j * c_tile + qq * 128 + lax.broadcasted_iota(
            jnp.int32, (128, b), 0)
        ge = jnp.where((sc >= tgt[...]) & (col < n), 1.0, 0.0)
        acc += jnp.sum(ge, axis=0, keepdims=True)

    @pl.when(j == 0)
    def _():
        rank_ref[...] = jnp.ones_like(rank_ref)

    rank_ref[...] += acc


def _main_call(q, tgt, ent, ling, vis, wl, wv, maskt, *, c_tile=1024,
               interpret=False):
    b, d = q.shape
    n = ent.shape[0]
    grid = (pl.cdiv(n, c_tile),)
    return pl.pallas_call(
        functools.partial(_main_body, c_tile, n),
        grid=grid,
        in_specs=[
            pl.BlockSpec((b, d), lambda j: (0, 0)),
            pl.BlockSpec((1, b), lambda j: (0, 0)),
            pl.BlockSpec((c_tile, d), lambda j: (j, 0)),
            pl.BlockSpec((c_tile, d), lambda j: (j, 0)),
            pl.BlockSpec((c_tile, d), lambda j: (j, 0)),
            pl.BlockSpec((d, d), lambda j: (0, 0)),
            pl.BlockSpec((d, d), lambda j: (0, 0)),
            pl.BlockSpec((1, 128, b), lambda j: (j // 4, 0, 0)),
        ],
        out_specs=[
            pl.BlockSpec((c_tile, b), lambda j: (j, 0)),
            pl.BlockSpec((1, b), lambda j: (0, 0)),
        ],
        out_shape=[
            jax.ShapeDtypeStruct((n, b), jnp.float32),
            jax.ShapeDtypeStruct((1, b), jnp.float32),
        ],
        interpret=interpret,
    )(q, tgt, ent, ling, vis, wl, wv, maskt)


def _mask_build(idx64, n_cols, *, interpret=False):
    b, kk = idx64.shape
    ngrp = kk // 16
    nrows = b // NUM_WORKERS
    ntiles = n_cols // 4096
    mesh = plsc.VectorSubcoreMesh(core_axis_name="c", subcore_axis_name="s")

    @functools.partial(
        pl.kernel,
        out_type=jax.ShapeDtypeStruct((ntiles, b, 128), jnp.int32),
        mesh=mesh,
        scratch_types=[
            pltpu.VMEM((n_cols,), jnp.int32),
            pltpu.VMEM((ntiles, 128), jnp.int32),
            pltpu.VMEM((kk,), jnp.int32),
        ],
        compiler_params=pltpu.CompilerParams(needs_layout_passes=False),
        interpret=interpret,
    )
    def k(idx_hbm, mask_hbm, elect, mrow, idxv):
        wid = lax.axis_index("s") * 2 + lax.axis_index("c")
        base = wid * nrows
        for step in range(nrows):
            row = base + step
            pltpu.sync_copy(idx_hbm.at[row], idxv)
            for g in range(ngrp):
                sl = pl.ds(g * 16, 16)
                ent = lax.iota(jnp.int32, 16) + g * 16
                plsc.store_scatter(elect, [idxv[sl]], ent)
            zero = jnp.zeros((16,), jnp.int32)
            for g in range(ntiles * 128 // 16):
                mrow[g * 16 // 128, pl.ds(g * 16 % 128, 16)] = zero
            for g in range(ngrp):
                sl = pl.ds(g * 16, 16)
                ent = lax.iota(jnp.int32, 16) + g * 16
                ii = idxv[sl]
                win = plsc.load_gather(elect, [ii]) == ent
                jhi = lax.shift_right_logical(ii, 12)
                ll = ii & 127
                p = lax.shift_right_logical(ii & 4095, 7)
                bit = lax.shift_left(jnp.ones((16,), jnp.int32), p)
                plsc.addupdate_scatter(mrow, [jhi, ll], bit, mask=win)
            pltpu.sync_copy(mrow, mask_hbm.at[:, row])

    return k(idx64)


def kernel(queries, ling, visual, filter_idx, ent_emb, rel_emb, W_ling, W_visual):
    b = queries.shape[0]
    n, d = ent_emb.shape
    f = filter_idx.shape[1]
    h = queries[:, 0].astype(jnp.int32)
    r = queries[:, 1].astype(jnp.int32)
    t = queries[:, 2].astype(jnp.int32)
    ht = jnp.concatenate([h, t])

    kk = -(-(f + 1) // 16) * 16
    idx64 = jnp.concatenate(
        [filter_idx.astype(jnp.int32),
         jnp.broadcast_to(t[:, None], (b, kk - f))], axis=1)

    c_tile = 1024
    n_pad4 = pl.cdiv(n, 4096) * 4096
    maskw = _mask_build(idx64, n_pad4)
    maskt = jnp.swapaxes(maskw, 1, 2)

    er, lr, vr, rr = _gather_rows(ht, r, ent_emb, ling, visual, rel_emb)
    q, tgt = _prep_call(er, lr, vr, rr, W_ling, W_visual)
    filt_t, ranks2 = _main_call(q, tgt, ent_emb, ling, visual,
                                W_ling, W_visual, maskt, c_tile=c_tile)
    return filt_t.T, tgt.reshape(b, 1), ranks2.reshape(b)


# speedup vs baseline: 2.8558x; 1.1217x over previous
import functools

import jax
import jax.numpy as jnp
from jax import lax
from jax.experimental import pallas as pl
from jax.experimental.pallas import tpu as pltpu
from jax.experimental.pallas import tpu_sc as plsc

NEG = -1000000.0
NUM_WORKERS = 32


def _gather_rows(ht, r, ent_emb, ling, visual, rel_emb, *, interpret=False):
    nht_total = ht.shape[0]
    nr_total = r.shape[0]
    d = ent_emb.shape[1]
    nht = nht_total // NUM_WORKERS
    nr = nr_total // NUM_WORKERS
    mesh = plsc.VectorSubcoreMesh(core_axis_name="c", subcore_axis_name="s")

    @functools.partial(
        pl.kernel,
        out_type=[jax.ShapeDtypeStruct((nht_total, d), jnp.float32)] * 3
        + [jax.ShapeDtypeStruct((nr_total, d), jnp.float32)],
        mesh=mesh,
        scratch_types=[
            pltpu.VMEM((nht,), jnp.int32),
            pltpu.VMEM((nr,), jnp.int32),
            pltpu.VMEM((nht, d), jnp.float32),
            pltpu.VMEM((nr, d), jnp.float32),
            pltpu.SemaphoreType.DMA,
        ],
        interpret=interpret,
    )
    def k(ht_hbm, r_hbm, ent_hbm, ling_hbm, vis_hbm, rel_hbm,
          oe, ol, ov, orel, idxh, idxr, bufh, bufr, sem):
        wid = lax.axis_index("s") * 2 + lax.axis_index("c")
        bh = wid * nht
        br = wid * nr
        pltpu.sync_copy(ht_hbm.at[pl.ds(bh, nht)], idxh)
        pltpu.sync_copy(r_hbm.at[pl.ds(br, nr)], idxr)
        for table, out in ((ent_hbm, oe), (ling_hbm, ol), (vis_hbm, ov)):
            pltpu.async_copy(table.at[idxh], bufh, sem).wait()
            pltpu.sync_copy(bufh, out.at[pl.ds(bh, nht)])
        pltpu.async_copy(rel_hbm.at[idxr], bufr, sem).wait()
        pltpu.sync_copy(bufr, orel.at[pl.ds(br, nr)])

    return k(ht, r, ent_emb, ling, visual, rel_emb)


def _prep_body(er, lr, vr, rr, wl, wv, q_ref, tgt_ref):
    b = rr.shape[0]
    fused = (
        er[...]
        + jnp.dot(lr[...], wl[...], preferred_element_type=jnp.float32)
        + jnp.dot(vr[...], wv[...], preferred_element_type=jnp.float32)
    )
    q = fused[:b, :] * rr[...]
    q_ref[...] = q
    tgt_ref[...] = jnp.sum(q * fused[b:, :], axis=1)[None, :]


def _prep_call(er, lr, vr, rr, wl, wv, *, interpret=False):
    b, d = rr.shape
    return pl.pallas_call(
        _prep_body,
        out_shape=[
            jax.ShapeDtypeStruct((b, d), jnp.float32),
            jax.ShapeDtypeStruct((1, b), jnp.float32),
        ],
        interpret=interpret,
    )(er, lr, vr, rr, wl, wv)


def _main_body(c_tile, n, q, tgt, ent, lingb, visb, wl, wv, mw, filt_ref, rank_ref):
    j = pl.program_id(0)
    b = q.shape[0]
    ae = (
        ent[...]
        + jnp.dot(lingb[...], wl[...], preferred_element_type=jnp.float32)
        + jnp.dot(visb[...], wv[...], preferred_element_type=jnp.float32)
    )
    s = lax.dot_general(ae, q[...], (((1,), (1,)), ((), ())),
                        preferred_element_type=jnp.float32)
    mw0 = mw[...][0]
    base_p = 8 * lax.rem(j, 4)
    acc = jnp.zeros((1, b), jnp.float32)
    for qq in range(c_tile // 128):
        sl = slice(qq * 128, (qq + 1) * 128)
        cm = (jnp.right_shift(mw0, base_p + qq) & 1) == 1
        sc = jnp.where(cm, NEG, s[sl, :])
        filt_ref[sl, :] = sc
        col = j * c_tile + qq * 128 + lax.broadcasted_iota(
            jnp.int32, (128, b), 0)
        ge = jnp.where((sc >= tgt[...]) & (col < n), 1.0, 0.0)
        acc += jnp.sum(ge, axis=0, keepdims=True)

    @pl.when(j == 0)
    def _():
        rank_ref[...] = jnp.ones_like(rank_ref)

    rank_ref[...] += acc


def _main_call(q, tgt, ent, ling, vis, wl, wv, maskt, *, c_tile=1024,
               interpret=False):
    b, d = q.shape
    n = ent.shape[0]
    grid = (pl.cdiv(n, c_tile),)
    return pl.pallas_call(
        functools.partial(_main_body, c_tile, n),
        grid=grid,
        in_specs=[
            pl.BlockSpec((b, d), lambda j: (0, 0)),
            pl.BlockSpec((1, b), lambda j: (0, 0)),
            pl.BlockSpec((c_tile, d), lambda j: (j, 0)),
            pl.BlockSpec((c_tile, d), lambda j: (j, 0)),
            pl.BlockSpec((c_tile, d), lambda j: (j, 0)),
            pl.BlockSpec((d, d), lambda j: (0, 0)),
            pl.BlockSpec((d, d), lambda j: (0, 0)),
            pl.BlockSpec((1, 128, b), lambda j: (j // 4, 0, 0)),
        ],
        out_specs=[
            pl.BlockSpec((c_tile, b), lambda j: (j, 0)),
            pl.BlockSpec((1, b), lambda j: (0, 0)),
        ],
        out_shape=[
            jax.ShapeDtypeStruct((n, b), jnp.float32),
            jax.ShapeDtypeStruct((1, b), jnp.float32),
        ],
        interpret=interpret,
    )(q, tgt, ent, ling, vis, wl, wv, maskt)


def _mask_build(idx64, n_cols, *, interpret=False):
    b, kk = idx64.shape
    ngrp = kk // 16
    nrows = b // NUM_WORKERS
    ntiles = n_cols // 4096
    mesh = plsc.VectorSubcoreMesh(core_axis_name="c", subcore_axis_name="s")

    zeros = jnp.zeros((ntiles, nrows, 128), jnp.int32)

    @functools.partial(
        pl.kernel,
        out_type=jax.ShapeDtypeStruct((ntiles, b, 128), jnp.int32),
        mesh=mesh,
        scratch_types=[
            pltpu.VMEM((n_cols,), jnp.int32),
            pltpu.VMEM((ntiles, nrows, 128), jnp.int32),
            pltpu.VMEM((nrows, kk), jnp.int32),
        ],
        compiler_params=pltpu.CompilerParams(needs_layout_passes=False),
        interpret=interpret,
    )
    def k(idx_hbm, zero_hbm, mask_hbm, elect, mrows, idxv):
        wid = lax.axis_index("s") * 2 + lax.axis_index("c")
        base = wid * nrows
        pltpu.sync_copy(idx_hbm.at[pl.ds(base, nrows)], idxv)
        pltpu.sync_copy(zero_hbm, mrows)
        for step in range(nrows):
            for g in range(ngrp):
                sl = pl.ds(g * 16, 16)
                ent = lax.iota(jnp.int32, 16) + g * 16
                plsc.store_scatter(elect, [idxv[step, sl]], ent)
            for g in range(ngrp):
                sl = pl.ds(g * 16, 16)
                ent = lax.iota(jnp.int32, 16) + g * 16
                ii = idxv[step, sl]
                win = plsc.load_gather(elect, [ii]) == ent
                jhi = lax.shift_right_logical(ii, 12)
                ll = ii & 127
                p = lax.shift_right_logical(ii & 4095, 7)
                bit = lax.shift_left(jnp.ones((16,), jnp.int32), p)
                rowv = jnp.full((16,), step, jnp.int32)
                plsc.addupdate_scatter(mrows, [jhi, rowv, ll], bit, mask=win)
        pltpu.sync_copy(mrows, mask_hbm.at[:, pl.ds(base, nrows)])

    return k(idx64, zeros)


def kernel(queries, ling, visual, filter_idx, ent_emb, rel_emb, W_ling, W_visual):
    b = queries.shape[0]
    n, d = ent_emb.shape
    f = filter_idx.shape[1]
    h = queries[:, 0].astype(jnp.int32)
    r = queries[:, 1].astype(jnp.int32)
    t = queries[:, 2].astype(jnp.int32)
    ht = jnp.concatenate([h, t])

    kk = -(-(f + 1) // 16) * 16
    idx64 = jnp.concatenate(
        [filter_idx.astype(jnp.int32),
         jnp.broadcast_to(t[:, None], (b, kk - f))], axis=1)

    c_tile = 1024
    n_pad4 = pl.cdiv(n, 4096) * 4096
    maskt = jnp.swapaxes(_mask_build(idx64, n_pad4), 1, 2)

    er, lr, vr, rr = _gather_rows(ht, r, ent_emb, ling, visual, rel_emb)
    q, tgt = _prep_call(er, lr, vr, rr, W_ling, W_visual)
    filt_t, ranks2 = _main_call(q, tgt, ent_emb, ling, visual,
                                W_ling, W_visual, maskt, c_tile=c_tile)
    return filt_t.T, tgt.reshape(b, 1), ranks2.reshape(b)


# speedup vs baseline: 3.1764x; 1.1123x over previous
import functools

import jax
import jax.numpy as jnp
from jax import lax
from jax.experimental import pallas as pl
from jax.experimental.pallas import tpu as pltpu
from jax.experimental.pallas import tpu_sc as plsc

NEG = -1000000.0
NUM_WORKERS = 32


def _gather_rows(ht, r, ent_emb, ling, visual, rel_emb, *, interpret=False):
    nht_total = ht.shape[0]
    nr_total = r.shape[0]
    d = ent_emb.shape[1]
    nht = nht_total // NUM_WORKERS
    nr = nr_total // NUM_WORKERS
    mesh = plsc.VectorSubcoreMesh(core_axis_name="c", subcore_axis_name="s")

    @functools.partial(
        pl.kernel,
        out_type=[jax.ShapeDtypeStruct((nht_total, d), jnp.float32)] * 3
        + [jax.ShapeDtypeStruct((nr_total, d), jnp.float32)],
        mesh=mesh,
        scratch_types=[
            pltpu.VMEM((nht,), jnp.int32),
            pltpu.VMEM((nr,), jnp.int32),
            pltpu.VMEM((nht, d), jnp.float32),
            pltpu.VMEM((nr, d), jnp.float32),
            pltpu.SemaphoreType.DMA,
        ],
        interpret=interpret,
    )
    def k(ht_hbm, r_hbm, ent_hbm, ling_hbm, vis_hbm, rel_hbm,
          oe, ol, ov, orel, idxh, idxr, bufh, bufr, sem):
        wid = lax.axis_index("s") * 2 + lax.axis_index("c")
        bh = wid * nht
        br = wid * nr
        pltpu.sync_copy(ht_hbm.at[pl.ds(bh, nht)], idxh)
        pltpu.sync_copy(r_hbm.at[pl.ds(br, nr)], idxr)
        for table, out in ((ent_hbm, oe), (ling_hbm, ol), (vis_hbm, ov)):
            pltpu.async_copy(table.at[idxh], bufh, sem).wait()
            pltpu.sync_copy(bufh, out.at[pl.ds(bh, nht)])
        pltpu.async_copy(rel_hbm.at[idxr], bufr, sem).wait()
        pltpu.sync_copy(bufr, orel.at[pl.ds(br, nr)])

    return k(ht, r, ent_emb, ling, visual, rel_emb)


def _prep_body(er, lr, vr, rr, wl, wv, q_ref, tgt_ref):
    b = rr.shape[0]
    fused = (
        er[...]
        + jnp.dot(lr[...], wl[...], preferred_element_type=jnp.float32)
        + jnp.dot(vr[...], wv[...], preferred_element_type=jnp.float32)
    )
    q = fused[:b, :] * rr[...]
    q_ref[...] = q
    tgt_ref[...] = jnp.sum(q * fused[b:, :], axis=1)[None, :]


def _prep_call(er, lr, vr, rr, wl, wv, *, interpret=False):
    b, d = rr.shape
    return pl.pallas_call(
        _prep_body,
        out_shape=[
            jax.ShapeDtypeStruct((b, d), jnp.float32),
            jax.ShapeDtypeStruct((1, b), jnp.float32),
        ],
        interpret=interpret,
    )(er, lr, vr, rr, wl, wv)


def _main_body(c_tile, n, q, tgt, ent, lingb, visb, wl, wv, mw, filt_ref, rank_ref):
    j = pl.program_id(0)
    b = q.shape[0]
    ae = (
        ent[...]
        + jnp.dot(lingb[...], wl[...], preferred_element_type=jnp.float32)
        + jnp.dot(visb[...], wv[...], preferred_element_type=jnp.float32)
    )
    s = lax.dot_general(ae, q[...], (((1,), (1,)), ((), ())),
                        preferred_element_type=jnp.float32)
    mw0 = mw[...][0]
    base_p = (c_tile // 128) * lax.rem(j, 4096 // c_tile)
    acc = jnp.zeros((1, b), jnp.float32)
    for qq in range(c_tile // 128):
        sl = slice(qq * 128, (qq + 1) * 128)
        cm = (jnp.right_shift(mw0, base_p + qq) & 1) == 1
        sc = jnp.where(cm, NEG, s[sl, :])
        filt_ref[sl, :] = sc
        col = j * c_tile + qq * 128 + lax.broadcasted_iota(
            jnp.int32, (128, b), 0)
        ge = jnp.where((sc >= tgt[...]) & (col < n), 1.0, 0.0)
        acc += jnp.sum(ge, axis=0, keepdims=True)

    @pl.when(j == 0)
    def _():
        rank_ref[...] = jnp.ones_like(rank_ref)

    rank_ref[...] += acc


def _main_call(q, tgt, ent, ling, vis, wl, wv, maskt, *, c_tile=1024,
               interpret=False):
    b, d = q.shape
    n = ent.shape[0]
    grid = (pl.cdiv(n, c_tile),)
    return pl.pallas_call(
        functools.partial(_main_body, c_tile, n),
        grid=grid,
        in_specs=[
            pl.BlockSpec((b, d), lambda j: (0, 0)),
            pl.BlockSpec((1, b), lambda j: (0, 0)),
            pl.BlockSpec((c_tile, d), lambda j: (j, 0)),
            pl.BlockSpec((c_tile, d), lambda j: (j, 0)),
            pl.BlockSpec((c_tile, d), lambda j: (j, 0)),
            pl.BlockSpec((d, d), lambda j: (0, 0)),
            pl.BlockSpec((d, d), lambda j: (0, 0)),
            pl.BlockSpec((1, 128, b),
                         lambda j, _q=4096 // c_tile: (j // _q, 0, 0)),
        ],
        out_specs=[
            pl.BlockSpec((c_tile, b), lambda j: (j, 0)),
            pl.BlockSpec((1, b), lambda j: (0, 0)),
        ],
        out_shape=[
            jax.ShapeDtypeStruct((n, b), jnp.float32),
            jax.ShapeDtypeStruct((1, b), jnp.float32),
        ],
        interpret=interpret,
    )(q, tgt, ent, ling, vis, wl, wv, maskt)


def _mask_build(idx64, n_cols, *, interpret=False):
    b, kk = idx64.shape
    ngrp = kk // 16
    nrows = b // NUM_WORKERS
    ntiles = n_cols // 4096
    mesh = plsc.VectorSubcoreMesh(core_axis_name="c", subcore_axis_name="s")

    zeros = jnp.zeros((ntiles, nrows, 128), jnp.int32)

    @functools.partial(
        pl.kernel,
        out_type=jax.ShapeDtypeStruct((ntiles, b, 128), jnp.int32),
        mesh=mesh,
        scratch_types=[
            pltpu.VMEM((n_cols,), jnp.int32),
            pltpu.VMEM((ntiles, nrows, 128), jnp.int32),
            pltpu.VMEM((nrows, kk), jnp.int32),
        ],
        compiler_params=pltpu.CompilerParams(needs_layout_passes=False),
        interpret=interpret,
    )
    def k(idx_hbm, zero_hbm, mask_hbm, elect, mrows, idxv):
        wid = lax.axis_index("s") * 2 + lax.axis_index("c")
        base = wid * nrows
        pltpu.sync_copy(idx_hbm.at[pl.ds(base, nrows)], idxv)
        pltpu.sync_copy(zero_hbm, mrows)
        for step in range(nrows):
            for g in range(ngrp):
                sl = pl.ds(g * 16, 16)
                ent = lax.iota(jnp.int32, 16) + g * 16
                plsc.store_scatter(elect, [idxv[step, sl]], ent)
            for g in range(ngrp):
                sl = pl.ds(g * 16, 16)
                ent = lax.iota(jnp.int32, 16) + g * 16
                ii = idxv[step, sl]
                win = plsc.load_gather(elect, [ii]) == ent
                jhi = lax.shift_right_logical(ii, 12)
                ll = ii & 127
                p = lax.shift_right_logical(ii & 4095, 7)
                bit = lax.shift_left(jnp.ones((16,), jnp.int32), p)
                rowv = jnp.full((16,), step, jnp.int32)
                plsc.addupdate_scatter(mrows, [jhi, rowv, ll], bit, mask=win)
        pltpu.sync_copy(mrows, mask_hbm.at[:, pl.ds(base, nrows)])

    return k(idx64, zeros)


def kernel(queries, ling, visual, filter_idx, ent_emb, rel_emb, W_ling, W_visual):
    b = queries.shape[0]
    n, d = ent_emb.shape
    f = filter_idx.shape[1]
    h = queries[:, 0].astype(jnp.int32)
    r = queries[:, 1].astype(jnp.int32)
    t = queries[:, 2].astype(jnp.int32)
    ht = jnp.concatenate([h, t])

    kk = -(-(f + 1) // 16) * 16
    idx64 = jnp.concatenate(
        [filter_idx.astype(jnp.int32),
         jnp.broadcast_to(t[:, None], (b, kk - f))], axis=1)

    c_tile = 2048
    n_pad4 = pl.cdiv(n, 4096) * 4096
    maskt = jnp.swapaxes(_mask_build(idx64, n_pad4), 1, 2)

    er, lr, vr, rr = _gather_rows(ht, r, ent_emb, ling, visual, rel_emb)
    q, tgt = _prep_call(er, lr, vr, rr, W_ling, W_visual)
    filt_t, ranks2 = _main_call(q, tgt, ent_emb, ling, visual,
                                W_ling, W_visual, maskt, c_tile=c_tile)
    return filt_t.T, tgt.reshape(b, 1), ranks2.reshape(b)


# speedup vs baseline: 3.2086x; 1.0101x over previous
import functools

import jax
import jax.numpy as jnp
from jax import lax
from jax.experimental import pallas as pl
from jax.experimental.pallas import tpu as pltpu
from jax.experimental.pallas import tpu_sc as plsc

NEG = -1000000.0
NUM_WORKERS = 32


def _gather_rows(ht, r, ent_emb, ling, visual, rel_emb, *, interpret=False):
    nht_total = ht.shape[0]
    nr_total = r.shape[0]
    d = ent_emb.shape[1]
    nht = nht_total // NUM_WORKERS
    nr = nr_total // NUM_WORKERS
    mesh = plsc.VectorSubcoreMesh(core_axis_name="c", subcore_axis_name="s")

    @functools.partial(
        pl.kernel,
        out_type=[jax.ShapeDtypeStruct((nht_total, d), jnp.float32)] * 3
        + [jax.ShapeDtypeStruct((nr_total, d), jnp.float32)],
        mesh=mesh,
        scratch_types=[
            pltpu.VMEM((nht,), jnp.int32),
            pltpu.VMEM((nr,), jnp.int32),
            pltpu.VMEM((nht, d), jnp.float32),
            pltpu.VMEM((nr, d), jnp.float32),
            pltpu.SemaphoreType.DMA,
        ],
        interpret=interpret,
    )
    def k(ht_hbm, r_hbm, ent_hbm, ling_hbm, vis_hbm, rel_hbm,
          oe, ol, ov, orel, idxh, idxr, bufh, bufr, sem):
        wid = lax.axis_index("s") * 2 + lax.axis_index("c")
        bh = wid * nht
        br = wid * nr
        pltpu.sync_copy(ht_hbm.at[pl.ds(bh, nht)], idxh)
        pltpu.sync_copy(r_hbm.at[pl.ds(br, nr)], idxr)
        for table, out in ((ent_hbm, oe), (ling_hbm, ol), (vis_hbm, ov)):
            pltpu.async_copy(table.at[idxh], bufh, sem).wait()
            pltpu.sync_copy(bufh, out.at[pl.ds(bh, nht)])
        pltpu.async_copy(rel_hbm.at[idxr], bufr, sem).wait()
        pltpu.sync_copy(bufr, orel.at[pl.ds(br, nr)])

    return k(ht, r, ent_emb, ling, visual, rel_emb)


def _prep_body(er, lr, vr, rr, wl, wv, q_ref, tgt_ref):
    b = rr.shape[0]
    fused = (
        er[...]
        + jnp.dot(lr[...], wl[...], preferred_element_type=jnp.float32)
        + jnp.dot(vr[...], wv[...], preferred_element_type=jnp.float32)
    )
    q = fused[:b, :] * rr[...]
    q_ref[...] = q
    tgt_ref[...] = jnp.sum(q * fused[b:, :], axis=1)[None, :]


def _prep_call(er, lr, vr, rr, wl, wv, *, interpret=False):
    b, d = rr.shape
    return pl.pallas_call(
        _prep_body,
        out_shape=[
            jax.ShapeDtypeStruct((b, d), jnp.float32),
            jax.ShapeDtypeStruct((1, b), jnp.float32),
        ],
        interpret=interpret,
    )(er, lr, vr, rr, wl, wv)


def _main_body(c_tile, n, q, tgt, ent, lingb, visb, wl, wv, mw, filt_ref, rank_ref):
    j = pl.program_id(0)
    b = q.shape[0]
    ae = (
        ent[...]
        + jnp.dot(lingb[...], wl[...], preferred_element_type=jnp.float32)
        + jnp.dot(visb[...], wv[...], preferred_element_type=jnp.float32)
    )
    s = lax.dot_general(ae, q[...], (((1,), (1,)), ((), ())),
                        preferred_element_type=jnp.float32)
    mw0 = mw[...][0]
    base_p = (c_tile // 128) * lax.rem(j, 4096 // c_tile)
    acc = jnp.zeros((1, b), jnp.float32)
    for qq in range(c_tile // 128):
        sl = slice(qq * 128, (qq + 1) * 128)
        cm = (jnp.right_shift(mw0, base_p + qq) & 1) == 1
        sc = jnp.where(cm, NEG, s[sl, :])
        filt_ref[sl, :] = sc
        col = j * c_tile + qq * 128 + lax.broadcasted_iota(
            jnp.int32, (128, b), 0)
        ge = jnp.where((sc >= tgt[...]) & (col < n), 1.0, 0.0)
        acc += jnp.sum(ge, axis=0, keepdims=True)

    @pl.when(j == 0)
    def _():
        rank_ref[...] = jnp.ones_like(rank_ref)

    rank_ref[...] += acc


def _main_call(q, tgt, ent, ling, vis, wl, wv, maskt, *, c_tile=1024,
               interpret=False):
    b, d = q.shape
    n = ent.shape[0]
    grid = (pl.cdiv(n, c_tile),)
    return pl.pallas_call(
        functools.partial(_main_body, c_tile, n),
        grid=grid,
        in_specs=[
            pl.BlockSpec((b, d), lambda j: (0, 0)),
            pl.BlockSpec((1, b), lambda j: (0, 0)),
            pl.BlockSpec((c_tile, d), lambda j: (j, 0)),
            pl.BlockSpec((c_tile, d), lambda j: (j, 0)),
            pl.BlockSpec((c_tile, d), lambda j: (j, 0)),
            pl.BlockSpec((d, d), lambda j: (0, 0)),
            pl.BlockSpec((d, d), lambda j: (0, 0)),
            pl.BlockSpec((1, 128, b),
                         lambda j, _q=4096 // c_tile: (j // _q, 0, 0)),
        ],
        out_specs=[
            pl.BlockSpec((c_tile, b), lambda j: (j, 0)),
            pl.BlockSpec((1, b), lambda j: (0, 0)),
        ],
        out_shape=[
            jax.ShapeDtypeStruct((n, b), jnp.float32),
            jax.ShapeDtypeStruct((1, b), jnp.float32),
        ],
        interpret=interpret,
    )(q, tgt, ent, ling, vis, wl, wv, maskt)


def _mask_build(idx64, n_cols, *, interpret=False):
    b, kk = idx64.shape
    ngrp = kk // 16
    nrows = b // NUM_WORKERS
    ntiles = n_cols // 4096
    mesh = plsc.VectorSubcoreMesh(core_axis_name="c", subcore_axis_name="s")

    zeros = jnp.zeros((ntiles, nrows, 128), jnp.int32)

    @functools.partial(
        pl.kernel,
        out_type=jax.ShapeDtypeStruct((ntiles, b, 128), jnp.int32),
        mesh=mesh,
        scratch_types=[
            pltpu.VMEM((n_cols,), jnp.int32),
            pltpu.VMEM((ntiles, nrows, 128), jnp.int32),
            pltpu.VMEM((nrows, kk), jnp.int32),
        ],
        compiler_params=pltpu.CompilerParams(needs_layout_passes=False),
        interpret=interpret,
    )
    def k(idx_hbm, zero_hbm, mask_hbm, elect, mrows, idxv):
        wid = lax.axis_index("s") * 2 + lax.axis_index("c")
        base = wid * nrows
        pltpu.sync_copy(idx_hbm.at[pl.ds(base, nrows)], idxv)
        pltpu.sync_copy(zero_hbm, mrows)
        for step in range(nrows):
            for g in range(ngrp):
                sl = pl.ds(g * 16, 16)
                ent = lax.iota(jnp.int32, 16) + g * 16
                plsc.store_scatter(elect, [idxv[step, sl]], ent)
            for g in range(ngrp):
                sl = pl.ds(g * 16, 16)
                ent = lax.iota(jnp.int32, 16) + g * 16
                ii = idxv[step, sl]
                win = plsc.load_gather(elect, [ii]) == ent
                jhi = lax.shift_right_logical(ii, 12)
                ll = ii & 127
                p = lax.shift_right_logical(ii & 4095, 7)
                bit = lax.shift_left(jnp.ones((16,), jnp.int32), p)
                rowv = jnp.full((16,), step, jnp.int32)
                plsc.addupdate_scatter(mrows, [jhi, rowv, ll], bit, mask=win)
        pltpu.sync_copy(mrows, mask_hbm.at[:, pl.ds(base, nrows)])

    return k(idx64, zeros)


def kernel(queries, ling, visual, filter_idx, ent_emb, rel_emb, W_ling, W_visual):
    b = queries.shape[0]
    n, d = ent_emb.shape
    f = filter_idx.shape[1]
    h = queries[:, 0].astype(jnp.int32)
    r = queries[:, 1].astype(jnp.int32)
    t = queries[:, 2].astype(jnp.int32)
    ht = jnp.concatenate([h, t])

    kk = -(-(f + 1) // 16) * 16
    idx64 = jnp.concatenate(
        [filter_idx.astype(jnp.int32),
         jnp.broadcast_to(t[:, None], (b, kk - f))], axis=1)

    c_tile = 4096
    n_pad4 = pl.cdiv(n, 4096) * 4096
    maskt = jnp.swapaxes(_mask_build(idx64, n_pad4), 1, 2)

    er, lr, vr, rr = _gather_rows(ht, r, ent_emb, ling, visual, rel_emb)
    q, tgt = _prep_call(er, lr, vr, rr, W_ling, W_visual)
    filt_t, ranks2 = _main_call(q, tgt, ent_emb, ling, visual,
                                W_ling, W_visual, maskt, c_tile=c_tile)
    return filt_t.T, tgt.reshape(b, 1), ranks2.reshape(b)
